# Initial kernel scaffold; baseline (speedup 1.0000x reference)
import jax
import jax.numpy as jnp
from jax.experimental import pallas as pl


def kernel(prediction, target):
    raise NotImplementedError("write your pallas kernel here")


# speedup vs baseline: 2.9323x; 2.9323x over previous
import functools

import jax
import jax.numpy as jnp
from jax import lax
from jax.experimental import pallas as pl
from jax.experimental.pallas import tpu as pltpu

_SMOOTH = 0.1


def _tc_body(x_ref, t_ref, out_ref, *, block_rows, classes):
    i = pl.program_id(0)
    x = x_ref[...]
    m = jnp.max(x, axis=1, keepdims=True)
    se = jnp.sum(jnp.exp(x - m), axis=1)
    lse = m[:, 0] + jnp.log(se)
    sx = jnp.sum(x, axis=1)

    t = t_ref[0, 0, :]
    col = lax.broadcasted_iota(jnp.int32, (block_rows, classes), 1)
    hit = jnp.where(col == t[:, None], x, 0.0)
    xt = jnp.sum(hit, axis=1)

    a = _SMOOTH / (classes - 1)
    b = (1.0 - _SMOOTH) - a
    part = jnp.sum((a * classes + b) * lse - a * sx - b * xt)

    @pl.when(i == 0)
    def _init():
        out_ref[0, 0] = 0.0

    out_ref[0, 0] += part


def kernel(prediction, target):
    n, classes = prediction.shape
    block_rows = 512
    grid = n // block_rows
    tgt = target.astype(jnp.int32).reshape(grid, 1, block_rows)

    total = pl.pallas_call(
        functools.partial(_tc_body, block_rows=block_rows, classes=classes),
        grid=(grid,),
        in_specs=[
            pl.BlockSpec((block_rows, classes), lambda i: (i, 0)),
            pl.BlockSpec((1, 1, block_rows), lambda i: (i, 0, 0)),
        ],
        out_specs=pl.BlockSpec(
            (1, 1), lambda i: (0, 0), memory_space=pltpu.SMEM
        ),
        out_shape=jax.ShapeDtypeStruct((1, 1), jnp.float32),
    )(prediction, tgt)

    return total[0, 0] / n
